# baseline (speedup 1.0000x reference)
import functools

import jax
import jax.numpy as jnp
from jax import lax
from jax.experimental import pallas as pl
from jax.experimental.pallas import tpu as pltpu
from jax.experimental.pallas import tpu_sc as plsc

N = 10000
E = 320000
D = 128
L = 16
CE = 128
NCHUNK = E // CE
NC = 2
NS = 16
NW = NC * NS
NP = 10240
RPT = NP // NS
PUB = 624
NEG_SLOPE = 0.2


def _mm_body(x_ref, ws_ref, wd_ref, hs_ref, hd_ref):
    xb = x_ref[...]
    hs_ref[...] = jnp.dot(xb, ws_ref[...], preferred_element_type=jnp.float32)
    hd_ref[...] = jnp.dot(xb, wd_ref[...], preferred_element_type=jnp.float32)


def _project(x, W_src, W_dst):
    bm = 1000
    return pl.pallas_call(
        _mm_body,
        grid=(N // bm,),
        in_specs=[
            pl.BlockSpec((bm, D), lambda i: (i, 0)),
            pl.BlockSpec((D, D), lambda i: (0, 0)),
            pl.BlockSpec((D, D), lambda i: (0, 0)),
        ],
        out_specs=[
            pl.BlockSpec((bm, D), lambda i: (i, 0)),
            pl.BlockSpec((bm, D), lambda i: (i, 0)),
        ],
        out_shape=[
            jax.ShapeDtypeStruct((N, D), jnp.float32),
            jax.ShapeDtypeStruct((N, D), jnp.float32),
        ],
    )(x, W_src, W_dst)


def _sc_edge_kernel(hs_hbm, hd_hbm, src_hbm, dst_hbm, att_hbm,
                    out_hbm, den_hbm,
                    sidx, didx, hsb, hdb, attb, den, S, sem1, sem2):
    cid = lax.axis_index("c")
    sid = lax.axis_index("s")
    wid = sid * NC + cid

    pltpu.sync_copy(att_hbm, attb)
    attv = [attb[pl.ds(j * L, L)] for j in range(D // L)]

    def _zden(r, _):
        den[pl.ds(r * L, L)] = jnp.zeros((L,), jnp.float32)
        return 0
    lax.fori_loop(0, NP // L, _zden, 0)

    def _zrow(r, _):
        for j in range(D // L):
            hsb[r, pl.ds(j * L, L)] = jnp.zeros((L,), jnp.float32)
        return 0
    lax.fori_loop(0, CE, _zrow, 0)
    row0 = pl.multiple_of(sid * RPT, CE)
    for i in range(RPT // CE):
        pltpu.sync_copy(hsb, S.at[pl.ds(row0 + i * CE, CE)])
    plsc.subcore_barrier()

    zero16 = jnp.zeros((L,), jnp.float32)
    lane_iota = lax.iota(jnp.int32, L)
    lane0 = lane_iota == 0

    def _edge(c):
        hsv = []
        acc = zero16
        for j in range(D // L):
            hs_j = hsb[c, pl.ds(j * L, L)]
            hd_j = hdb[c, pl.ds(j * L, L)]
            hsv.append(hs_j)
            v = hs_j + hd_j
            lrelu = jnp.maximum(v, NEG_SLOPE * v)
            acc = acc + lrelu * attv[j]
        for k in (8, 4, 2, 1):
            perm = lane_iota ^ k
            acc = acc + jnp.take_along_axis(
                acc, perm, axis=0, mode="promise_in_bounds")
        exv = jnp.exp(acc)
        for j in range(D // L):
            hsb[c, pl.ds(j * L, L)] = hsv[j] * exv
        dvec = plsc.load_gather(didx, [jnp.full((L,), 0, jnp.int32) + c])
        plsc.addupdate_scatter(den, [dvec], exv, mask=lane0)

    nch = (NCHUNK - wid + NW - 1) // NW

    def _chunk(t, _):
        base = (wid + t * NW) * CE
        pltpu.sync_copy(src_hbm.at[pl.ds(base, CE)], sidx)
        pltpu.sync_copy(dst_hbm.at[pl.ds(base, CE)], didx)
        cp1 = pltpu.async_copy(hs_hbm.at[sidx], hsb, sem1)
        cp2 = pltpu.async_copy(hd_hbm.at[didx], hdb, sem2)
        cp1.wait()
        cp2.wait()
        plsc.parallel_loop(0, CE, unroll=4)(_edge)
        pltpu.sync_copy(hsb, S.at[didx], add=True)
        return 0

    lax.fori_loop(0, nch, _chunk, 0)
    plsc.subcore_barrier()

    @pl.when(sid < NS - 1)
    def _pub_main():
        b = pl.multiple_of(sid * PUB, 8)
        pltpu.sync_copy(S.at[pl.ds(b, PUB)], out_hbm.at[cid, pl.ds(b, PUB)])

    @pl.when(sid == NS - 1)
    def _pub_last():
        b = (NS - 1) * PUB
        nlast = N - b
        pltpu.sync_copy(S.at[pl.ds(b, nlast)], out_hbm.at[cid, pl.ds(b, nlast)])

    pltpu.sync_copy(den, den_hbm.at[cid, sid])


def _sc_edge(hs, hd, src, dst, att):
    mesh = plsc.VectorSubcoreMesh(core_axis_name="c", subcore_axis_name="s")
    f = functools.partial(
        pl.kernel,
        out_type=(
            jax.ShapeDtypeStruct((NC, N, D), jnp.float32),
            jax.ShapeDtypeStruct((NC, NS, NP), jnp.float32),
        ),
        mesh=mesh,
        compiler_params=pltpu.CompilerParams(needs_layout_passes=False),
        scratch_types=[
            pltpu.VMEM((CE,), jnp.int32),
            pltpu.VMEM((CE,), jnp.int32),
            pltpu.VMEM((CE, D), jnp.float32),
            pltpu.VMEM((CE, D), jnp.float32),
            pltpu.VMEM((D,), jnp.float32),
            pltpu.VMEM((NP,), jnp.float32),
            pltpu.VMEM_SHARED((NP, D), jnp.float32),
            pltpu.SemaphoreType.DMA,
            pltpu.SemaphoreType.DMA,
        ],
    )(_sc_edge_kernel)
    return f(hs, hd, src, dst, att)


def _combine_body(s_ref, d_ref, b_ref, o_ref):
    s = s_ref[...]
    tot = s[0] + s[1]
    ones = jnp.ones((NW, D), jnp.float32)
    den = jnp.dot(d_ref[...], ones, preferred_element_type=jnp.float32)
    o_ref[...] = tot / (den + 1e-16) + b_ref[...]


def _combine(Sext, den_t, bias2d):
    bm = 1000
    return pl.pallas_call(
        _combine_body,
        grid=(N // bm,),
        in_specs=[
            pl.BlockSpec((NC, bm, D), lambda i: (0, i, 0)),
            pl.BlockSpec((bm, NW), lambda i: (i, 0)),
            pl.BlockSpec((1, D), lambda i: (0, 0)),
        ],
        out_specs=pl.BlockSpec((bm, D), lambda i: (i, 0)),
        out_shape=jax.ShapeDtypeStruct((N, D), jnp.float32),
    )(Sext, den_t, bias2d)


def kernel(x, edge_index, W_src, W_dst, att, bias):
    hs, hd = _project(x, W_src, W_dst)
    src = edge_index[0]
    dst = edge_index[1]
    Sext, den = _sc_edge(hs, hd, src, dst, att)
    den_t = den.reshape(NW, NP)[:, :N].T
    return _combine(Sext, den_t, bias.reshape(1, D))


# speedup vs baseline: 1.1981x; 1.1981x over previous
import functools

import jax
import jax.numpy as jnp
from jax import lax
from jax.experimental import pallas as pl
from jax.experimental.pallas import tpu as pltpu
from jax.experimental.pallas import tpu_sc as plsc

N = 10000
E = 320000
D = 128
L = 16
CE = 64
NCHUNK = E // CE
NC = 2
NS = 16
NW = NC * NS
NP = 10240
RPT = NP // NS
PUB = 624
NEG_SLOPE = 0.2


def _mm_body(x_ref, ws_ref, wd_ref, hs_ref, hd_ref):
    xb = x_ref[...]
    hs_ref[...] = jnp.dot(xb, ws_ref[...], preferred_element_type=jnp.float32)
    hd_ref[...] = jnp.dot(xb, wd_ref[...], preferred_element_type=jnp.float32)


def _project(x, W_src, W_dst):
    bm = 1000
    return pl.pallas_call(
        _mm_body,
        grid=(N // bm,),
        in_specs=[
            pl.BlockSpec((bm, D), lambda i: (i, 0)),
            pl.BlockSpec((D, D), lambda i: (0, 0)),
            pl.BlockSpec((D, D), lambda i: (0, 0)),
        ],
        out_specs=[
            pl.BlockSpec((bm, D), lambda i: (i, 0)),
            pl.BlockSpec((bm, D), lambda i: (i, 0)),
        ],
        out_shape=[
            jax.ShapeDtypeStruct((N, D), jnp.float32),
            jax.ShapeDtypeStruct((N, D), jnp.float32),
        ],
    )(x, W_src, W_dst)


def _sc_edge_kernel(hs_hbm, hd_hbm, src_hbm, dst_hbm, att_hbm,
                    out_hbm, den_hbm,
                    sidx0, sidx1, didx0, didx1, hsb0, hsb1, hdb0, hdb1,
                    attb, den, S, semg0, semg1, sems0, sems1):
    cid = lax.axis_index("c")
    sid = lax.axis_index("s")
    wid = sid * NC + cid
    sidx = (sidx0, sidx1)
    didx = (didx0, didx1)
    hsb = (hsb0, hsb1)
    hdb = (hdb0, hdb1)
    semg = (semg0, semg1)
    sems = (sems0, sems1)

    pltpu.sync_copy(att_hbm, attb)
    attv = [attb[pl.ds(j * L, L)] for j in range(D // L)]

    def _zden(r, _):
        den[pl.ds(r * L, L)] = jnp.zeros((L,), jnp.float32)
        return 0
    lax.fori_loop(0, NP // L, _zden, 0)

    def _zrow(r, _):
        for j in range(D // L):
            hsb0[r, pl.ds(j * L, L)] = jnp.zeros((L,), jnp.float32)
        return 0
    lax.fori_loop(0, CE, _zrow, 0)
    row0 = pl.multiple_of(sid * RPT, CE)
    for i in range(RPT // CE):
        pltpu.sync_copy(hsb0, S.at[pl.ds(row0 + i * CE, CE)])
    plsc.subcore_barrier()

    zero16 = jnp.zeros((L,), jnp.float32)
    lane_iota = lax.iota(jnp.int32, L)
    lane0 = lane_iota == 0

    def _make_edge(b):
        def _edge(c):
            hsv = []
            acc = zero16
            for j in range(D // L):
                hs_j = hsb[b][c, pl.ds(j * L, L)]
                hd_j = hdb[b][c, pl.ds(j * L, L)]
                hsv.append(hs_j)
                v = hs_j + hd_j
                lrelu = jnp.maximum(v, NEG_SLOPE * v)
                acc = acc + lrelu * attv[j]
            for k in (8, 4, 2, 1):
                perm = lane_iota ^ k
                acc = acc + jnp.take_along_axis(
                    acc, perm, axis=0, mode="promise_in_bounds")
            exv = jnp.exp(acc)
            for j in range(D // L):
                hsb[b][c, pl.ds(j * L, L)] = hsv[j] * exv
            dvec = plsc.load_gather(
                didx[b], [jnp.full((L,), 0, jnp.int32) + c])
            plsc.addupdate_scatter(den, [dvec], exv, mask=lane0)
        return _edge

    nch = (NCHUNK - wid + NW - 1) // NW

    def _prefetch(b, t):
        base = (wid + t * NW) * CE
        pltpu.sync_copy(src_hbm.at[pl.ds(base, CE)], sidx[b])
        pltpu.sync_copy(dst_hbm.at[pl.ds(base, CE)], didx[b])
        pltpu.async_copy(hs_hbm.at[sidx[b]], hsb[b], semg[b])
        pltpu.async_copy(hd_hbm.at[didx[b]], hdb[b], semg[b])

    _prefetch(0, jnp.int32(0))

    npairs = (nch + 1) // 2

    def _pair(p, _):
        for b in (0, 1):
            t = p * 2 + b
            nb = 1 - b

            @pl.when(t < nch)
            def _do():
                @pl.when(t + 1 < nch)
                def _pref():
                    @pl.when(t >= 1)
                    def _wsc():
                        pltpu.make_async_copy(
                            hsb[nb], S.at[didx[nb]], sems[nb]).wait()
                    _prefetch(nb, t + 1)

                pltpu.make_async_copy(
                    hs_hbm.at[sidx[b]], hsb[b], semg[b]).wait()
                pltpu.make_async_copy(
                    hd_hbm.at[didx[b]], hdb[b], semg[b]).wait()
                plsc.parallel_loop(0, CE, unroll=2)(_make_edge(b))
                pltpu.async_copy(hsb[b], S.at[didx[b]], sems[b], add=True)
        return 0

    lax.fori_loop(0, npairs, _pair, 0)
    for b in (0, 1):
        pltpu.make_async_copy(hsb[b], S.at[didx[b]], sems[b]).wait()
    plsc.subcore_barrier()

    @pl.when(sid < NS - 1)
    def _pub_main():
        b = pl.multiple_of(sid * PUB, 8)
        pltpu.sync_copy(S.at[pl.ds(b, PUB)], out_hbm.at[cid, pl.ds(b, PUB)])

    @pl.when(sid == NS - 1)
    def _pub_last():
        b = (NS - 1) * PUB
        nlast = N - b
        pltpu.sync_copy(S.at[pl.ds(b, nlast)], out_hbm.at[cid, pl.ds(b, nlast)])

    pltpu.sync_copy(den, den_hbm.at[cid, sid])


def _sc_edge(hs, hd, src, dst, att):
    mesh = plsc.VectorSubcoreMesh(core_axis_name="c", subcore_axis_name="s")
    f = functools.partial(
        pl.kernel,
        out_type=(
            jax.ShapeDtypeStruct((NC, N, D), jnp.float32),
            jax.ShapeDtypeStruct((NC, NS, NP), jnp.float32),
        ),
        mesh=mesh,
        compiler_params=pltpu.CompilerParams(needs_layout_passes=False),
        scratch_types=[
            pltpu.VMEM((CE,), jnp.int32),
            pltpu.VMEM((CE,), jnp.int32),
            pltpu.VMEM((CE,), jnp.int32),
            pltpu.VMEM((CE,), jnp.int32),
            pltpu.VMEM((CE, D), jnp.float32),
            pltpu.VMEM((CE, D), jnp.float32),
            pltpu.VMEM((CE, D), jnp.float32),
            pltpu.VMEM((CE, D), jnp.float32),
            pltpu.VMEM((D,), jnp.float32),
            pltpu.VMEM((NP,), jnp.float32),
            pltpu.VMEM_SHARED((NP, D), jnp.float32),
            pltpu.SemaphoreType.DMA,
            pltpu.SemaphoreType.DMA,
            pltpu.SemaphoreType.DMA,
            pltpu.SemaphoreType.DMA,
        ],
    )(_sc_edge_kernel)
    return f(hs, hd, src, dst, att)


def _combine_body(s_ref, d_ref, b_ref, o_ref):
    s = s_ref[...]
    tot = s[0] + s[1]
    ones = jnp.ones((NW, D), jnp.float32)
    den = jnp.dot(d_ref[...], ones, preferred_element_type=jnp.float32)
    o_ref[...] = tot / (den + 1e-16) + b_ref[...]


def _combine(Sext, den_t, bias2d):
    bm = 1000
    return pl.pallas_call(
        _combine_body,
        grid=(N // bm,),
        in_specs=[
            pl.BlockSpec((NC, bm, D), lambda i: (0, i, 0)),
            pl.BlockSpec((bm, NW), lambda i: (i, 0)),
            pl.BlockSpec((1, D), lambda i: (0, 0)),
        ],
        out_specs=pl.BlockSpec((bm, D), lambda i: (i, 0)),
        out_shape=jax.ShapeDtypeStruct((N, D), jnp.float32),
    )(Sext, den_t, bias2d)


def kernel(x, edge_index, W_src, W_dst, att, bias):
    hs, hd = _project(x, W_src, W_dst)
    src = edge_index[0]
    dst = edge_index[1]
    Sext, den = _sc_edge(hs, hd, src, dst, att)
    den_t = den.reshape(NW, NP)[:, :N].T
    return _combine(Sext, den_t, bias.reshape(1, D))
